# baseline (speedup 1.0000x reference)
import jax
import jax.numpy as jnp
from jax import lax
from jax.experimental import pallas as pl
from jax.experimental.pallas import tpu as pltpu
from jax.experimental.pallas import tpu_sc as plsc

B = 16384
F = 26
EMB = 16
NUM_IN = 1000012
NW = 32
ROWS_W = B // NW
CB = 64
NCHUNK = ROWS_W // CB
IDX_PER_CHUNK = CB * F
IDX_W = 128
STREAMS = IDX_PER_CHUNK // IDX_W
IDX_ROWS_W = ROWS_W * F // IDX_W

PW = 512
NPANEL = NUM_IN // PW
TAIL_BASE = NPANEL * PW
OUT_ROWS = (NPANEL * PW * EMB + 2048) // 128
MAIN_IT = 61

_MESH = dict(core_axis_name="c", subcore_axis_name="s")


def _wid():
    return lax.axis_index("s") * 2 + lax.axis_index("c")


def _tr_body(embT_hbm, tail_hbm, out_hbm,
             pan0, pan1, ov0, ov1, pt, si0, si1, so0, so1):
    w = _wid()
    lane = jnp.arange(16, dtype=jnp.int32)
    pans, ovs = (pan0, pan1), (ov0, ov1)
    sis, sos = (si0, si1), (so0, so1)
    OB = PW * EMB // 128

    def fire_in(p, slot):
        return pltpu.async_copy(
            embT_hbm.at[:, pl.ds(pl.multiple_of(p * PW, PW), PW)],
            pans[slot], sis[slot])

    def wait_in(slot):
        pltpu.make_async_copy(
            embT_hbm.at[:, pl.ds(0, PW)], pans[slot], sis[slot]).wait()

    def fire_out(p, slot):
        return pltpu.async_copy(
            ovs[slot], out_hbm.at[pl.ds(p * OB, OB)], sos[slot])

    def wait_out(slot):
        pltpu.make_async_copy(
            ovs[slot], out_hbm.at[pl.ds(0, OB)], sos[slot]).wait()

    def transpose_panel(pan, ov, n_groups):
        for k in range(16):
            m = (lane + k) & 15
            a = m >> 3
            b1 = ((m & 7) << 4) + lane

            def gbody(g4, carry, m=m, a=a, b1=b1):
                for dk in range(4):
                    g = g4 * 4 + dk
                    v = plsc.load_gather(pan, [lane, m + g * 16])
                    plsc.store_scatter(ov, [a + g * 2, b1], v)
                return carry
            lax.fori_loop(0, n_groups // 4, gbody, None)

    fire_in(w, 0)

    def body(i, carry):
        p = w + 32 * i
        live = p < NPANEL
        nxt = (p + 32) < NPANEL
        for s in (0, 1):
            @pl.when(((i % 2) == s) & live)
            def _(s=s, p=p, i=i, nxt=nxt):
                wait_in(s)

                @pl.when(nxt)
                def _():
                    fire_in(p + 32, 1 - s)

                @pl.when(i >= 2)
                def _():
                    wait_out(s)

                transpose_panel(pans[s], ovs[s], PW // 16)
                fire_out(p, s)
        return carry

    lax.fori_loop(0, MAIN_IT + 1, body, None)
    wait_out(0)
    wait_out(1)

    @pl.when(w == 0)
    def _():
        pltpu.async_copy(tail_hbm, pt, si0).wait()
        pltpu.async_copy(pt, out_hbm.at[pl.ds(NPANEL * OB, 16)], so0).wait()


_tr = pl.kernel(
    _tr_body,
    out_type=jax.ShapeDtypeStruct((OUT_ROWS, 128), jnp.float32),
    mesh=plsc.VectorSubcoreMesh(**_MESH),
    scratch_types=[
        pltpu.VMEM((EMB, PW), jnp.float32),
        pltpu.VMEM((EMB, PW), jnp.float32),
        pltpu.VMEM((PW // 8, 128), jnp.float32),
        pltpu.VMEM((PW // 8, 128), jnp.float32),
        pltpu.VMEM((EMB, 128), jnp.float32),
        pltpu.SemaphoreType.DMA,
        pltpu.SemaphoreType.DMA,
        pltpu.SemaphoreType.DMA,
        pltpu.SemaphoreType.DMA,
    ],
    compiler_params=pltpu.CompilerParams(
        needs_layout_passes=False, use_tc_tiling_on_sc=True),
)


def _fm_body(x_hbm, emb_hbm, pw_hbm, fcp_hbm, out_hbm,
             idx_v, rows_v, pv_v, out_v, fc_v, sem):
    wid = _wid()
    pltpu.sync_copy(x_hbm.at[pl.ds(wid * IDX_ROWS_W, IDX_ROWS_W)], idx_v)
    pltpu.sync_copy(fcp_hbm, fc_v)
    fcv = fc_v[...]
    w = fcv[0]
    bias = fcv[1]
    lane = jnp.arange(16, dtype=jnp.int32)

    def chunk_body(c, carry0):
        handles = []
        for j in range(STREAMS):
            jr = c * STREAMS + j
            handles.append(pltpu.async_copy(
                emb_hbm.at[idx_v.at[jr]],
                rows_v.at[pl.ds(j * IDX_W, IDX_W)], sem))
            handles.append(pltpu.async_copy(
                pw_hbm.at[idx_v.at[jr]],
                pv_v.at[pl.ds(j * IDX_W, IDX_W)], sem))
        for h in handles:
            h.wait()

        def grp_body(g, carry1):
            zvec = jnp.zeros((16,), jnp.float32)
            pacc = jnp.zeros((16,), jnp.float32)
            pbase = g * (16 * F) + lane * F
            for f in range(F):
                pacc = pacc + plsc.load_gather(pv_v, [pbase + f])
            for i in range(16):
                e0 = (g * 16 + i) * F
                v = rows_v[e0]
                s = v
                q = v * v
                for f in range(1, F):
                    v = rows_v[e0 + f]
                    s = s + v
                    q = q + v * v
                z = jnp.sum(0.5 * (s * s - q))
                zvec = jnp.where(lane == i, z, zvec)
            logit = (zvec + pacc) * w + bias
            out_v[pl.ds(c * CB + g * 16, 16)] = 1.0 / (1.0 + jnp.exp(-logit))
            return carry1

        lax.fori_loop(0, CB // 16, grp_body, None)
        return carry0

    lax.fori_loop(0, NCHUNK, chunk_body, None)
    pltpu.sync_copy(out_v, out_hbm.at[pl.ds(wid * ROWS_W, ROWS_W)])


_fm = pl.kernel(
    _fm_body,
    out_type=jax.ShapeDtypeStruct((B,), jnp.float32),
    mesh=plsc.VectorSubcoreMesh(**_MESH),
    scratch_types=[
        pltpu.VMEM((IDX_ROWS_W, IDX_W), jnp.int32),
        pltpu.VMEM((IDX_PER_CHUNK, EMB), jnp.float32),
        pltpu.VMEM((IDX_PER_CHUNK,), jnp.float32),
        pltpu.VMEM((ROWS_W,), jnp.float32),
        pltpu.VMEM((16,), jnp.float32),
        pltpu.SemaphoreType.DMA,
    ],
    compiler_params=pltpu.CompilerParams(
        needs_layout_passes=False, use_tc_tiling_on_sc=False),
)


@jax.jit
def kernel(x, emb, proj_w, fc_w, fc_b):
    embT = emb.T
    tail = jnp.pad(emb[TAIL_BASE:], ((0, 128 - (NUM_IN - TAIL_BASE)), (0, 0)))
    tail = tail.reshape(16, 128)
    table = _tr(embT, tail).reshape(OUT_ROWS * 128 // EMB, EMB)
    x2 = x.reshape(B * F // IDX_W, IDX_W)
    fcp = jnp.pad(jnp.concatenate([fc_w.reshape(1), fc_b]), (0, 14))
    return _fm(x2, table, proj_w.reshape(-1), fcp.astype(jnp.float32))


# speedup vs baseline: 1.1085x; 1.1085x over previous
import jax
import jax.numpy as jnp
from jax import lax
from jax.experimental import pallas as pl
from jax.experimental.pallas import tpu as pltpu
from jax.experimental.pallas import tpu_sc as plsc

B = 16384
F = 26
EMB = 16
NUM_IN = 1000012
NW = 32
ROWS_W = B // NW
CB = 64
NCHUNK = ROWS_W // CB
IDX_PER_CHUNK = CB * F
IDX_W = 128
STREAMS = IDX_PER_CHUNK // IDX_W
IDX_ROWS_W = ROWS_W * F // IDX_W

PW = 1024
NPANEL = NUM_IN // PW
TAIL_BASE = NPANEL * PW
OB = PW * EMB // 128
OUT_ROWS = 125008
TAIL_OUT = OUT_ROWS - NPANEL * OB
MAIN_IT = 31

_MESH = dict(core_axis_name="c", subcore_axis_name="s")


def _wid():
    return lax.axis_index("s") * 2 + lax.axis_index("c")


def _tr_body(embT_hbm, tail_hbm, out_hbm,
             pan0, pan1, ov0, ov1, pt, si0, si1, so0, so1):
    w = _wid()
    lane = jnp.arange(16, dtype=jnp.int32)
    pans, ovs = (pan0, pan1), (ov0, ov1)
    sis, sos = (si0, si1), (so0, so1)

    def fire_in(p, slot):
        return pltpu.async_copy(
            embT_hbm.at[:, pl.ds(pl.multiple_of(p * PW, PW), PW)],
            pans[slot], sis[slot])

    def wait_in(slot):
        pltpu.make_async_copy(
            embT_hbm.at[:, pl.ds(0, PW)], pans[slot], sis[slot]).wait()

    def fire_out(p, slot):
        return pltpu.async_copy(
            ovs[slot], out_hbm.at[pl.ds(p * OB, OB)], sos[slot])

    def wait_out(slot):
        pltpu.make_async_copy(
            ovs[slot], out_hbm.at[pl.ds(0, OB)], sos[slot]).wait()

    def transpose_panel(pan, ov, n_groups):
        def gbody(g, carry):
            rl0 = jnp.zeros((16,), jnp.int32) + g * 16
            g0 = jnp.zeros((16,), jnp.int32) + g * 2
            for k in range(16):
                m = (lane + k) & 15
                v = plsc.load_gather(pan, [lane, rl0 + m])
                idx0 = g0 + (m >> 3)
                idx1 = ((m & 7) << 4) + lane
                plsc.store_scatter(ov, [idx0, idx1], v)
            return carry
        lax.fori_loop(0, n_groups, gbody, None)

    fire_in(w, 0)

    def body(i, carry):
        p = w + 32 * i
        live = p < NPANEL
        nxt = (p + 32) < NPANEL
        for s in (0, 1):
            @pl.when(((i % 2) == s) & live)
            def _(s=s, p=p, i=i, nxt=nxt):
                wait_in(s)

                @pl.when(nxt)
                def _():
                    fire_in(p + 32, 1 - s)

                @pl.when(i >= 2)
                def _():
                    wait_out(s)

                transpose_panel(pans[s], ovs[s], PW // 16)
                fire_out(p, s)
        return carry

    lax.fori_loop(0, MAIN_IT, body, None)
    wait_out(0)
    wait_out(1)

    @pl.when(w == 0)
    def _():
        pltpu.async_copy(tail_hbm, pt, si0).wait()
        pltpu.async_copy(pt, out_hbm.at[pl.ds(NPANEL * OB, TAIL_OUT)], so0).wait()


_tr = pl.kernel(
    _tr_body,
    out_type=jax.ShapeDtypeStruct((OUT_ROWS, 128), jnp.float32),
    mesh=plsc.VectorSubcoreMesh(**_MESH),
    scratch_types=[
        pltpu.VMEM((EMB, PW), jnp.float32),
        pltpu.VMEM((EMB, PW), jnp.float32),
        pltpu.VMEM((PW // 8, 128), jnp.float32),
        pltpu.VMEM((PW // 8, 128), jnp.float32),
        pltpu.VMEM((TAIL_OUT, 128), jnp.float32),
        pltpu.SemaphoreType.DMA,
        pltpu.SemaphoreType.DMA,
        pltpu.SemaphoreType.DMA,
        pltpu.SemaphoreType.DMA,
    ],
    compiler_params=pltpu.CompilerParams(
        needs_layout_passes=False, use_tc_tiling_on_sc=True),
)


def _fm_body(x_hbm, emb_hbm, pw_hbm, fcp_hbm, out_hbm,
             idx_v, rows_v, pv_v, out_v, fc_v, sem):
    wid = _wid()
    pltpu.sync_copy(x_hbm.at[pl.ds(wid * IDX_ROWS_W, IDX_ROWS_W)], idx_v)
    pltpu.sync_copy(fcp_hbm, fc_v)
    fcv = fc_v[...]
    w = fcv[0]
    bias = fcv[1]
    lane = jnp.arange(16, dtype=jnp.int32)

    def chunk_body(c, carry0):
        handles = []
        for j in range(STREAMS):
            jr = c * STREAMS + j
            handles.append(pltpu.async_copy(
                emb_hbm.at[idx_v.at[jr]],
                rows_v.at[pl.ds(j * IDX_W, IDX_W)], sem))
            handles.append(pltpu.async_copy(
                pw_hbm.at[idx_v.at[jr]],
                pv_v.at[pl.ds(j * IDX_W, IDX_W)], sem))
        for h in handles:
            h.wait()

        def grp_body(g, carry1):
            zvec = jnp.zeros((16,), jnp.float32)
            pacc = jnp.zeros((16,), jnp.float32)
            pbase = g * (16 * F) + lane * F
            for f in range(F):
                pacc = pacc + plsc.load_gather(pv_v, [pbase + f])
            for i in range(16):
                e0 = (g * 16 + i) * F
                v = rows_v[e0]
                s = v
                q = v * v
                for f in range(1, F):
                    v = rows_v[e0 + f]
                    s = s + v
                    q = q + v * v
                z = jnp.sum(0.5 * (s * s - q))
                zvec = jnp.where(lane == i, z, zvec)
            logit = (zvec + pacc) * w + bias
            out_v[pl.ds(c * CB + g * 16, 16)] = 1.0 / (1.0 + jnp.exp(-logit))
            return carry1

        lax.fori_loop(0, CB // 16, grp_body, None)
        return carry0

    lax.fori_loop(0, NCHUNK, chunk_body, None)
    pltpu.sync_copy(out_v, out_hbm.at[pl.ds(wid * ROWS_W, ROWS_W)])


_fm = pl.kernel(
    _fm_body,
    out_type=jax.ShapeDtypeStruct((B,), jnp.float32),
    mesh=plsc.VectorSubcoreMesh(**_MESH),
    scratch_types=[
        pltpu.VMEM((IDX_ROWS_W, IDX_W), jnp.int32),
        pltpu.VMEM((IDX_PER_CHUNK, EMB), jnp.float32),
        pltpu.VMEM((IDX_PER_CHUNK,), jnp.float32),
        pltpu.VMEM((ROWS_W,), jnp.float32),
        pltpu.VMEM((16,), jnp.float32),
        pltpu.SemaphoreType.DMA,
    ],
    compiler_params=pltpu.CompilerParams(
        needs_layout_passes=False, use_tc_tiling_on_sc=False),
)


@jax.jit
def kernel(x, emb, proj_w, fc_w, fc_b):
    embT = emb.T
    tail = jnp.pad(emb[TAIL_BASE:],
                   ((0, TAIL_OUT * 8 - (NUM_IN - TAIL_BASE)), (0, 0)))
    tail = tail.reshape(TAIL_OUT, 128)
    table = _tr(embT, tail).reshape(OUT_ROWS * 128 // EMB, EMB)
    x2 = x.reshape(B * F // IDX_W, IDX_W)
    fcp = jnp.pad(jnp.concatenate([fc_w.reshape(1), fc_b]), (0, 14))
    return _fm(x2, table, proj_w.reshape(-1), fcp.astype(jnp.float32))


# speedup vs baseline: 1.2220x; 1.1024x over previous
import jax
import jax.numpy as jnp
from jax import lax
from jax.experimental import pallas as pl
from jax.experimental.pallas import tpu as pltpu
from jax.experimental.pallas import tpu_sc as plsc

B = 16384
F = 26
EMB = 16
NUM_IN = 1000012
NW = 32
ROWS_W = B // NW
CB = 64
NCHUNK = ROWS_W // CB
IDX_PER_CHUNK = CB * F
IDX_W = 128
STREAMS = IDX_PER_CHUNK // IDX_W
IDX_ROWS_W = ROWS_W * F // IDX_W

PW = 1024
NPANEL = NUM_IN // PW
TAIL_BASE = NPANEL * PW
OB = PW * EMB // 128
OUT_ROWS = 125008
TAIL_OUT = OUT_ROWS - NPANEL * OB
MAIN_IT = 31

_MESH = dict(core_axis_name="c", subcore_axis_name="s")


def _wid():
    return lax.axis_index("s") * 2 + lax.axis_index("c")


def _tr_body(embT_hbm, tail_hbm, out_hbm,
             pan0, pan1, ov0, ov1, pt, si0, si1, so0, so1):
    w = _wid()
    lane = jnp.arange(16, dtype=jnp.int32)
    pans, ovs = (pan0, pan1), (ov0, ov1)
    sis, sos = (si0, si1), (so0, so1)

    def fire_in(p, slot):
        return pltpu.async_copy(
            embT_hbm.at[:, pl.ds(pl.multiple_of(p * PW, PW), PW)],
            pans[slot], sis[slot])

    def wait_in(slot):
        pltpu.make_async_copy(
            embT_hbm.at[:, pl.ds(0, PW)], pans[slot], sis[slot]).wait()

    def fire_out(p, slot):
        return pltpu.async_copy(
            ovs[slot], out_hbm.at[pl.ds(p * OB, OB)], sos[slot])

    def wait_out(slot):
        pltpu.make_async_copy(
            ovs[slot], out_hbm.at[pl.ds(0, OB)], sos[slot]).wait()

    def transpose_panel(pan, ov, n_groups):
        def gbody(g, carry):
            m2 = lane + g * 16
            idx0 = m2 >> 3
            base1 = (m2 & 7) << 4
            for d in range(16):
                v = pan[d, pl.ds(g * 16, 16)]
                plsc.store_scatter(ov, [idx0, base1 + d], v)
            return carry
        lax.fori_loop(0, n_groups, gbody, None)

    fire_in(w, 0)

    def body(i, carry):
        p = w + 32 * i
        live = p < NPANEL
        nxt = (p + 32) < NPANEL
        for s in (0, 1):
            @pl.when(((i % 2) == s) & live)
            def _(s=s, p=p, i=i, nxt=nxt):
                wait_in(s)

                @pl.when(nxt)
                def _():
                    fire_in(p + 32, 1 - s)

                @pl.when(i >= 2)
                def _():
                    wait_out(s)

                transpose_panel(pans[s], ovs[s], PW // 16)
                fire_out(p, s)
        return carry

    lax.fori_loop(0, MAIN_IT, body, None)
    wait_out(0)
    wait_out(1)

    @pl.when(w == 0)
    def _():
        pltpu.async_copy(tail_hbm, pt, si0).wait()
        pltpu.async_copy(pt, out_hbm.at[pl.ds(NPANEL * OB, TAIL_OUT)], so0).wait()


_tr = pl.kernel(
    _tr_body,
    out_type=jax.ShapeDtypeStruct((OUT_ROWS, 128), jnp.float32),
    mesh=plsc.VectorSubcoreMesh(**_MESH),
    scratch_types=[
        pltpu.VMEM((EMB, PW), jnp.float32),
        pltpu.VMEM((EMB, PW), jnp.float32),
        pltpu.VMEM((PW // 8, 128), jnp.float32),
        pltpu.VMEM((PW // 8, 128), jnp.float32),
        pltpu.VMEM((TAIL_OUT, 128), jnp.float32),
        pltpu.SemaphoreType.DMA,
        pltpu.SemaphoreType.DMA,
        pltpu.SemaphoreType.DMA,
        pltpu.SemaphoreType.DMA,
    ],
    compiler_params=pltpu.CompilerParams(
        needs_layout_passes=False, use_tc_tiling_on_sc=True),
)


def _fm_body(x_hbm, emb_hbm, pw_hbm, fcp_hbm, out_hbm,
             idx_v, rows_v, pv_v, out_v, fc_v, sem):
    wid = _wid()
    pltpu.sync_copy(x_hbm.at[pl.ds(wid * IDX_ROWS_W, IDX_ROWS_W)], idx_v)
    pltpu.sync_copy(fcp_hbm, fc_v)
    fcv = fc_v[...]
    w = fcv[0]
    bias = fcv[1]
    lane = jnp.arange(16, dtype=jnp.int32)

    def chunk_body(c, carry0):
        handles = []
        for j in range(STREAMS):
            jr = c * STREAMS + j
            handles.append(pltpu.async_copy(
                emb_hbm.at[idx_v.at[jr]],
                rows_v.at[pl.ds(j * IDX_W, IDX_W)], sem))
            handles.append(pltpu.async_copy(
                pw_hbm.at[idx_v.at[jr]],
                pv_v.at[pl.ds(j * IDX_W, IDX_W)], sem))
        for h in handles:
            h.wait()

        def grp_body(g, carry1):
            zvec = jnp.zeros((16,), jnp.float32)
            pacc = jnp.zeros((16,), jnp.float32)
            pbase = g * (16 * F) + lane * F
            for f in range(F):
                pacc = pacc + plsc.load_gather(pv_v, [pbase + f])
            for i in range(16):
                e0 = (g * 16 + i) * F
                v = rows_v[e0]
                s = v
                q = v * v
                for f in range(1, F):
                    v = rows_v[e0 + f]
                    s = s + v
                    q = q + v * v
                z = jnp.sum(0.5 * (s * s - q))
                zvec = jnp.where(lane == i, z, zvec)
            logit = (zvec + pacc) * w + bias
            out_v[pl.ds(c * CB + g * 16, 16)] = 1.0 / (1.0 + jnp.exp(-logit))
            return carry1

        lax.fori_loop(0, CB // 16, grp_body, None)
        return carry0

    lax.fori_loop(0, NCHUNK, chunk_body, None)
    pltpu.sync_copy(out_v, out_hbm.at[pl.ds(wid * ROWS_W, ROWS_W)])


_fm = pl.kernel(
    _fm_body,
    out_type=jax.ShapeDtypeStruct((B,), jnp.float32),
    mesh=plsc.VectorSubcoreMesh(**_MESH),
    scratch_types=[
        pltpu.VMEM((IDX_ROWS_W, IDX_W), jnp.int32),
        pltpu.VMEM((IDX_PER_CHUNK, EMB), jnp.float32),
        pltpu.VMEM((IDX_PER_CHUNK,), jnp.float32),
        pltpu.VMEM((ROWS_W,), jnp.float32),
        pltpu.VMEM((16,), jnp.float32),
        pltpu.SemaphoreType.DMA,
    ],
    compiler_params=pltpu.CompilerParams(
        needs_layout_passes=False, use_tc_tiling_on_sc=False),
)


@jax.jit
def kernel(x, emb, proj_w, fc_w, fc_b):
    embT = emb.T
    tail = jnp.pad(emb[TAIL_BASE:],
                   ((0, TAIL_OUT * 8 - (NUM_IN - TAIL_BASE)), (0, 0)))
    tail = tail.reshape(TAIL_OUT, 128)
    table = _tr(embT, tail).reshape(OUT_ROWS * 128 // EMB, EMB)
    x2 = x.reshape(B * F // IDX_W, IDX_W)
    fcp = jnp.pad(jnp.concatenate([fc_w.reshape(1), fc_b]), (0, 14))
    return _fm(x2, table, proj_w.reshape(-1), fcp.astype(jnp.float32))


# speedup vs baseline: 1.3092x; 1.0713x over previous
import jax
import jax.numpy as jnp
from jax import lax
from jax.experimental import pallas as pl
from jax.experimental.pallas import tpu as pltpu
from jax.experimental.pallas import tpu_sc as plsc

B = 16384
F = 26
EMB = 16
NUM_IN = 1000012
NW = 32
ROWS_W = B // NW
CB = 64
NCHUNK = ROWS_W // CB
IDX_PER_CHUNK = CB * F
IDX_W = 128
STREAMS = IDX_PER_CHUNK // IDX_W
IDX_ROWS_W = ROWS_W * F // IDX_W

PW = 1024
NPANEL = NUM_IN // PW
TAIL_BASE = NPANEL * PW
OB = PW * EMB // 128
OUT_ROWS = 125008
TAIL_OUT = OUT_ROWS - NPANEL * OB
MAIN_IT = 31

_MESH = dict(core_axis_name="c", subcore_axis_name="s")


def _wid():
    return lax.axis_index("s") * 2 + lax.axis_index("c")


def _tr_body(embT_hbm, tail_hbm, out_hbm,
             pan0, pan1, ov0, ov1, pt, si0, si1, so0, so1):
    w = _wid()
    lane = jnp.arange(16, dtype=jnp.int32)
    pans, ovs = (pan0, pan1), (ov0, ov1)
    sis, sos = (si0, si1), (so0, so1)

    def fire_in(p, slot):
        return pltpu.async_copy(
            embT_hbm.at[:, pl.ds(pl.multiple_of(p * PW, PW), PW)],
            pans[slot], sis[slot])

    def wait_in(slot):
        pltpu.make_async_copy(
            embT_hbm.at[:, pl.ds(0, PW)], pans[slot], sis[slot]).wait()

    def fire_out(p, slot):
        return pltpu.async_copy(
            ovs[slot], out_hbm.at[pl.ds(p * OB, OB)], sos[slot])

    def wait_out(slot):
        pltpu.make_async_copy(
            ovs[slot], out_hbm.at[pl.ds(0, OB)], sos[slot]).wait()

    def transpose_panel(pan, ov, n_groups):
        def gbody(g, carry):
            m2 = lane + g * 16
            idx0 = m2 >> 3
            base1 = (m2 & 7) << 4
            for d in range(16):
                v = pan[d, pl.ds(g * 16, 16)]
                plsc.store_scatter(ov, [idx0, base1 + d], v)
            return carry
        lax.fori_loop(0, n_groups, gbody, None)

    fire_in(w, 0)

    def body(i, carry):
        p = w + 32 * i
        live = p < NPANEL
        nxt = (p + 32) < NPANEL
        for s in (0, 1):
            @pl.when(((i % 2) == s) & live)
            def _(s=s, p=p, i=i, nxt=nxt):
                wait_in(s)

                @pl.when(nxt)
                def _():
                    fire_in(p + 32, 1 - s)

                @pl.when(i >= 2)
                def _():
                    wait_out(s)

                transpose_panel(pans[s], ovs[s], PW // 16)
                fire_out(p, s)
        return carry

    lax.fori_loop(0, MAIN_IT, body, None)
    wait_out(0)
    wait_out(1)

    @pl.when(w == 0)
    def _():
        pltpu.async_copy(tail_hbm, pt, si0).wait()
        pltpu.async_copy(pt, out_hbm.at[pl.ds(NPANEL * OB, TAIL_OUT)], so0).wait()


_tr = pl.kernel(
    _tr_body,
    out_type=jax.ShapeDtypeStruct((OUT_ROWS, 128), jnp.float32),
    mesh=plsc.VectorSubcoreMesh(**_MESH),
    scratch_types=[
        pltpu.VMEM((EMB, PW), jnp.float32),
        pltpu.VMEM((EMB, PW), jnp.float32),
        pltpu.VMEM((PW // 8, 128), jnp.float32),
        pltpu.VMEM((PW // 8, 128), jnp.float32),
        pltpu.VMEM((TAIL_OUT, 128), jnp.float32),
        pltpu.SemaphoreType.DMA,
        pltpu.SemaphoreType.DMA,
        pltpu.SemaphoreType.DMA,
        pltpu.SemaphoreType.DMA,
    ],
    compiler_params=pltpu.CompilerParams(
        needs_layout_passes=False, use_tc_tiling_on_sc=True),
)


def _fm_body(x_hbm, emb_hbm, pw_hbm, fcp_hbm, out_hbm,
             idx_v, rows0, rows1, pv0, pv1, out_v, fc_v, sg0, sg1):
    wid = _wid()
    pltpu.sync_copy(x_hbm.at[pl.ds(wid * IDX_ROWS_W, IDX_ROWS_W)], idx_v)
    pltpu.sync_copy(fcp_hbm, fc_v)
    fcv = fc_v[...]
    w = fcv[0]
    bias = fcv[1]
    lane = jnp.arange(16, dtype=jnp.int32)
    rows, pvs, sgs = (rows0, rows1), (pv0, pv1), (sg0, sg1)

    def fire(c, s):
        for j in range(STREAMS):
            jr = c * STREAMS + j
            pltpu.async_copy(emb_hbm.at[idx_v.at[jr]],
                             rows[s].at[pl.ds(j * IDX_W, IDX_W)], sgs[s])
            pltpu.async_copy(pw_hbm.at[idx_v.at[jr]],
                             pvs[s].at[pl.ds(j * IDX_W, IDX_W)], sgs[s])

    def drain(s):
        pltpu.make_async_copy(
            emb_hbm.at[pl.ds(0, IDX_PER_CHUNK)], rows[s], sgs[s]).wait()
        pltpu.make_async_copy(
            pw_hbm.at[pl.ds(0, IDX_PER_CHUNK)], pvs[s], sgs[s]).wait()

    def compute(c, s):
        rows_v, pv_v = rows[s], pvs[s]

        def grp_body(g, carry1):
            zvec = jnp.zeros((16,), jnp.float32)
            pacc = jnp.zeros((16,), jnp.float32)
            pbase = g * (16 * F) + lane * F
            for f in range(F):
                pacc = pacc + plsc.load_gather(pv_v, [pbase + f])
            for i in range(16):
                e0 = (g * 16 + i) * F
                v = rows_v[e0]
                s_ = v
                q = v * v
                for f in range(1, F):
                    v = rows_v[e0 + f]
                    s_ = s_ + v
                    q = q + v * v
                z = jnp.sum(0.5 * (s_ * s_ - q))
                zvec = jnp.where(lane == i, z, zvec)
            logit = (zvec + pacc) * w + bias
            out_v[pl.ds(c * CB + g * 16, 16)] = 1.0 / (1.0 + jnp.exp(-logit))
            return carry1

        lax.fori_loop(0, CB // 16, grp_body, None)

    fire(jnp.int32(0), 0)

    def chunk_body(c, carry0):
        for s in (0, 1):
            @pl.when((c % 2) == s)
            def _(c=c, s=s):
                @pl.when(c + 1 < NCHUNK)
                def _():
                    fire(c + 1, 1 - s)
                drain(s)
                compute(c, s)
        return carry0

    lax.fori_loop(0, NCHUNK, chunk_body, None)
    pltpu.sync_copy(out_v, out_hbm.at[pl.ds(wid * ROWS_W, ROWS_W)])


_fm = pl.kernel(
    _fm_body,
    out_type=jax.ShapeDtypeStruct((B,), jnp.float32),
    mesh=plsc.VectorSubcoreMesh(**_MESH),
    scratch_types=[
        pltpu.VMEM((IDX_ROWS_W, IDX_W), jnp.int32),
        pltpu.VMEM((IDX_PER_CHUNK, EMB), jnp.float32),
        pltpu.VMEM((IDX_PER_CHUNK, EMB), jnp.float32),
        pltpu.VMEM((IDX_PER_CHUNK,), jnp.float32),
        pltpu.VMEM((IDX_PER_CHUNK,), jnp.float32),
        pltpu.VMEM((ROWS_W,), jnp.float32),
        pltpu.VMEM((16,), jnp.float32),
        pltpu.SemaphoreType.DMA,
        pltpu.SemaphoreType.DMA,
    ],
    compiler_params=pltpu.CompilerParams(
        needs_layout_passes=False, use_tc_tiling_on_sc=False),
)


@jax.jit
def kernel(x, emb, proj_w, fc_w, fc_b):
    embT = emb.T
    tail = jnp.pad(emb[TAIL_BASE:],
                   ((0, TAIL_OUT * 8 - (NUM_IN - TAIL_BASE)), (0, 0)))
    tail = tail.reshape(TAIL_OUT, 128)
    table = _tr(embT, tail).reshape(OUT_ROWS * 128 // EMB, EMB)
    x2 = x.reshape(B * F // IDX_W, IDX_W)
    fcp = jnp.pad(jnp.concatenate([fc_w.reshape(1), fc_b]), (0, 14))
    return _fm(x2, table, proj_w.reshape(-1), fcp.astype(jnp.float32))


# speedup vs baseline: 1.8483x; 1.4118x over previous
import jax
import jax.numpy as jnp
from jax import lax
from jax.experimental import pallas as pl
from jax.experimental.pallas import tpu as pltpu
from jax.experimental.pallas import tpu_sc as plsc

B = 16384
F = 26
EMB = 16
NUM_IN = 1000012
NW = 32
ROWS_W = B // NW
CB = 64
NCHUNK = ROWS_W // CB
IDX_PER_CHUNK = CB * F
IDX_W = 128
STREAMS = IDX_PER_CHUNK // IDX_W
IDX_ROWS_W = ROWS_W * F // IDX_W

PW = 1024
NPANEL = NUM_IN // PW
TAIL_BASE = NPANEL * PW
OB = PW * EMB // 128
OUT_ROWS = 125008
TAIL_OUT = OUT_ROWS - NPANEL * OB
MAIN_IT = 31

_MESH = dict(core_axis_name="c", subcore_axis_name="s")


def _wid():
    return lax.axis_index("s") * 2 + lax.axis_index("c")


def _tr_body(embT_hbm, tail_hbm, out_hbm,
             pan0, pan1, ov0, ov1, pt, si0, si1, so0, so1):
    w = _wid()
    lane = jnp.arange(16, dtype=jnp.int32)
    pans, ovs = (pan0, pan1), (ov0, ov1)
    sis, sos = (si0, si1), (so0, so1)

    def fire_in(p, slot):
        return pltpu.async_copy(
            embT_hbm.at[:, pl.ds(pl.multiple_of(p * PW, PW), PW)],
            pans[slot], sis[slot])

    def wait_in(slot):
        pltpu.make_async_copy(
            embT_hbm.at[:, pl.ds(0, PW)], pans[slot], sis[slot]).wait()

    def fire_out(p, slot):
        return pltpu.async_copy(
            ovs[slot], out_hbm.at[pl.ds(p * OB, OB)], sos[slot])

    def wait_out(slot):
        pltpu.make_async_copy(
            ovs[slot], out_hbm.at[pl.ds(0, OB)], sos[slot]).wait()

    def transpose_panel(pan, ov, n_groups):
        def gbody(g, carry):
            m2 = lane + g * 16
            idx0 = m2 >> 3
            base1 = (m2 & 7) << 4
            for h in range(2):
                vs = [pan[h * 8 + j, pl.ds(g * 16, 16)] for j in range(8)]
                for j in range(8):
                    plsc.store_scatter(ov, [idx0, base1 + (h * 8 + j)], vs[j])
            return carry
        lax.fori_loop(0, n_groups, gbody, None)

    fire_in(w, 0)

    def body(i, carry):
        p = w + 32 * i
        live = p < NPANEL
        nxt = (p + 32) < NPANEL
        for s in (0, 1):
            @pl.when(((i % 2) == s) & live)
            def _(s=s, p=p, i=i, nxt=nxt):
                wait_in(s)

                @pl.when(nxt)
                def _():
                    fire_in(p + 32, 1 - s)

                @pl.when(i >= 2)
                def _():
                    wait_out(s)

                transpose_panel(pans[s], ovs[s], PW // 16)
                fire_out(p, s)
        return carry

    lax.fori_loop(0, MAIN_IT, body, None)
    wait_out(0)
    wait_out(1)

    @pl.when(w == 0)
    def _():
        pltpu.async_copy(tail_hbm, pt, si0).wait()
        pltpu.async_copy(pt, out_hbm.at[pl.ds(NPANEL * OB, TAIL_OUT)], so0).wait()


_tr = pl.kernel(
    _tr_body,
    out_type=jax.ShapeDtypeStruct((OUT_ROWS, 128), jnp.float32),
    mesh=plsc.VectorSubcoreMesh(**_MESH),
    scratch_types=[
        pltpu.VMEM((EMB, PW), jnp.float32),
        pltpu.VMEM((EMB, PW), jnp.float32),
        pltpu.VMEM((PW // 8, 128), jnp.float32),
        pltpu.VMEM((PW // 8, 128), jnp.float32),
        pltpu.VMEM((TAIL_OUT, 128), jnp.float32),
        pltpu.SemaphoreType.DMA,
        pltpu.SemaphoreType.DMA,
        pltpu.SemaphoreType.DMA,
        pltpu.SemaphoreType.DMA,
    ],
    compiler_params=pltpu.CompilerParams(
        needs_layout_passes=False, use_tc_tiling_on_sc=True),
)


def _fm_body(x_hbm, emb_hbm, pw_hbm, fcp_hbm, out_hbm,
             idx_v, rows0, rows1, pv0, pv1, out_v, fc_v, sg0, sg1):
    wid = _wid()
    pltpu.sync_copy(x_hbm.at[pl.ds(wid * IDX_ROWS_W, IDX_ROWS_W)], idx_v)
    pltpu.sync_copy(fcp_hbm, fc_v)
    fcv = fc_v[...]
    w = fcv[0]
    bias = fcv[1]
    lane = jnp.arange(16, dtype=jnp.int32)
    rows, pvs, sgs = (rows0, rows1), (pv0, pv1), (sg0, sg1)

    def fire(c, s):
        for j in range(STREAMS):
            jr = c * STREAMS + j
            pltpu.async_copy(emb_hbm.at[idx_v.at[jr]],
                             rows[s].at[pl.ds(j * IDX_W, IDX_W)], sgs[s])
            pltpu.async_copy(pw_hbm.at[idx_v.at[jr]],
                             pvs[s].at[pl.ds(j * IDX_W, IDX_W)], sgs[s])

    def drain(s):
        pltpu.make_async_copy(
            emb_hbm.at[pl.ds(0, IDX_PER_CHUNK)], rows[s], sgs[s]).wait()
        pltpu.make_async_copy(
            pw_hbm.at[pl.ds(0, IDX_PER_CHUNK)], pvs[s], sgs[s]).wait()

    def compute(c, s):
        rows_v, pv_v = rows[s], pvs[s]

        def grp_body(g, carry1):
            zvec = jnp.zeros((16,), jnp.float32)
            pacc = jnp.zeros((16,), jnp.float32)
            pbase = g * (16 * F) + lane * F
            for f in range(F):
                pacc = pacc + plsc.load_gather(pv_v, [pbase + f])
            for i in range(16):
                e0 = (g * 16 + i) * F
                v = rows_v[e0]
                s_ = v
                q = v * v
                for f in range(1, F):
                    v = rows_v[e0 + f]
                    s_ = s_ + v
                    q = q + v * v
                z = jnp.sum(0.5 * (s_ * s_ - q))
                zvec = jnp.where(lane == i, z, zvec)
            logit = (zvec + pacc) * w + bias
            out_v[pl.ds(c * CB + g * 16, 16)] = 1.0 / (1.0 + jnp.exp(-logit))
            return carry1

        lax.fori_loop(0, CB // 16, grp_body, None)

    fire(jnp.int32(0), 0)

    def chunk_body(c, carry0):
        for s in (0, 1):
            @pl.when((c % 2) == s)
            def _(c=c, s=s):
                @pl.when(c + 1 < NCHUNK)
                def _():
                    fire(c + 1, 1 - s)
                drain(s)
                compute(c, s)
        return carry0

    lax.fori_loop(0, NCHUNK, chunk_body, None)
    pltpu.sync_copy(out_v, out_hbm.at[pl.ds(wid * ROWS_W, ROWS_W)])


_fm = pl.kernel(
    _fm_body,
    out_type=jax.ShapeDtypeStruct((B,), jnp.float32),
    mesh=plsc.VectorSubcoreMesh(**_MESH),
    scratch_types=[
        pltpu.VMEM((IDX_ROWS_W, IDX_W), jnp.int32),
        pltpu.VMEM((IDX_PER_CHUNK, EMB), jnp.float32),
        pltpu.VMEM((IDX_PER_CHUNK, EMB), jnp.float32),
        pltpu.VMEM((IDX_PER_CHUNK,), jnp.float32),
        pltpu.VMEM((IDX_PER_CHUNK,), jnp.float32),
        pltpu.VMEM((ROWS_W,), jnp.float32),
        pltpu.VMEM((16,), jnp.float32),
        pltpu.SemaphoreType.DMA,
        pltpu.SemaphoreType.DMA,
    ],
    compiler_params=pltpu.CompilerParams(
        needs_layout_passes=False, use_tc_tiling_on_sc=False),
)


@jax.jit
def kernel(x, emb, proj_w, fc_w, fc_b):
    embT = emb.T
    tail = jnp.pad(emb[TAIL_BASE:],
                   ((0, TAIL_OUT * 8 - (NUM_IN - TAIL_BASE)), (0, 0)))
    tail = tail.reshape(TAIL_OUT, 128)
    table = _tr(embT, tail).reshape(OUT_ROWS * 128 // EMB, EMB)
    x2 = x.reshape(B * F // IDX_W, IDX_W)
    fcp = jnp.pad(jnp.concatenate([fc_w.reshape(1), fc_b]), (0, 14))
    return _fm(x2, table, proj_w.reshape(-1), fcp.astype(jnp.float32))
